# Initial kernel scaffold; baseline (speedup 1.0000x reference)
import jax
import jax.numpy as jnp
from jax.experimental import pallas as pl


def kernel(ban_ids, item_ids, aux1, aux2, aux3):
    raise NotImplementedError("write your pallas kernel here")


# speedup vs baseline: 9.2471x; 9.2471x over previous
import functools

import jax
import jax.numpy as jnp
from jax.experimental import pallas as pl
from jax.experimental.pallas import tpu as pltpu

_NUM_ITEMS = 200
_PAD = 0
_NEG = -1000000000.0


def _spop_block(item_ref, ban_ref, out_ref, *, S, K, C):
    Bn = item_ref.shape[0]
    items = item_ref[...]
    ban = ban_ref[...]

    col2 = jax.lax.broadcasted_iota(jnp.int32, (Bn, C), 1)

    counts = jnp.zeros((Bn, C), jnp.float32)
    last = jnp.zeros((Bn, 1), jnp.int32)
    for j in range(S):
        it = items[:, j : j + 1]
        valid = it != _PAD
        counts = counts + jnp.where((it == col2) & valid, 1.0, 0.0)
        last = jnp.where(valid, it, last)
    counts = counts - jnp.where((last == col2) & (last != _PAD), 1.0, 0.0)

    col3 = jax.lax.broadcasted_iota(jnp.int32, (Bn, S, C), 2)
    logits = jnp.broadcast_to(counts[:, None, :], (Bn, S, C))
    banned = (ban[:, :, 0:1] == col3)
    for k in range(1, K):
        banned = banned | (ban[:, :, k : k + 1] == col3)
    logits = jnp.where(banned, logits + _NEG, logits)

    se = jnp.sum(jnp.exp(logits), axis=-1, keepdims=True)
    out_ref[...] = logits - jnp.log(se)


def kernel(ban_ids, item_ids, aux1, aux2, aux3):
    del aux1, aux2, aux3
    N, S = item_ids.shape
    K = ban_ids.shape[-1]
    C = _NUM_ITEMS
    Bn = 128

    item_ids = item_ids.astype(jnp.int32)
    ban_ids = ban_ids.astype(jnp.int32)

    grid = (N // Bn,)
    pi = pl.pallas_call(
        functools.partial(_spop_block, S=S, K=K, C=C),
        grid=grid,
        in_specs=[
            pl.BlockSpec((Bn, S), lambda i: (i, 0)),
            pl.BlockSpec((Bn, S, K), lambda i: (i, 0, 0)),
        ],
        out_specs=pl.BlockSpec((Bn, S, C), lambda i: (i, 0, 0)),
        out_shape=jax.ShapeDtypeStruct((N, S, C), jnp.float32),
        compiler_params=pltpu.CompilerParams(
            dimension_semantics=("parallel",),
        ),
    )(item_ids, ban_ids)

    v = jnp.zeros((N, S, 1), jnp.float32)
    return (pi, v)
